# baseline (speedup 1.0000x reference)
import functools

import jax
import jax.numpy as jnp
from jax import lax
from jax.experimental import pallas as pl
from jax.experimental.pallas import tpu as pltpu
import jax.experimental.pallas.tpu_sc as plsc

N = 10000
NP = 10240
E = 320000
EDGES = E + N
C = 128
C3 = 64
NSUB = 16
NCORE = 2
NW = NCORE * NSUB
DEPTH = 3
CHUNKS_A = 84
EP = NW * C * CHUNKS_A
EPW = EP // NW
CHUNKS_A3 = EPW // C3
CHUNKS_B = EP // (NSUB * C)
EPB = EP // NSUB
ROWS_W = NP // NSUB
EPSILON = 1e-16
BM = 1280

_SC_PARAMS = pltpu.CompilerParams(needs_layout_passes=False,
                                  use_tc_tiling_on_sc=False)


def _zero_vmem_rows(buf, rows, width):
    zero16 = jnp.zeros((16,), jnp.float32)

    def _z(i, carry):
        for k in range(width // 16):
            buf[i, pl.ds(k * 16, 16)] = zero16
        return carry

    lax.fori_loop(0, rows, _z, 0)


def _make_a_body(feat, ck, chunks, epw):
    nsl = feat // 16
    ngr = ck // 16

    def _a_body(xl_h, xr_h, srcw_h, dstw_h, att_h, ex_h, s_h,
                siv, div,
                xlr0, xlr1, xlr2, xrr0, xrr1, xrr2, exv0, exv1, exv2,
                attb, s_sh, *sems):
        xlr = [xlr0, xlr1, xlr2]
        xrr = [xrr0, xrr1, xrr2]
        exv = [exv0, exv1, exv2]
        semxl = sems[0:3]
        semxr = sems[3:6]
        semex = sems[6:9]
        semss = sems[9:12]
        cc = lax.axis_index("c")
        ss = lax.axis_index("s")
        w = cc * NSUB + ss
        zero16 = jnp.zeros((16,), jnp.float32)

        pltpu.sync_copy(srcw_h.at[w], siv)
        pltpu.sync_copy(dstw_h.at[w], div)

        for k in range(ngr):
            exv0[pl.ds(k * 16, 16)] = zero16
        for k in range(ROWS_W // ck):
            pltpu.sync_copy(exv0, s_sh.at[pl.ds(ss * ROWS_W + k * ck, ck)])
        pltpu.sync_copy(att_h, attb)
        plsc.subcore_barrier()

        iota16 = lax.iota(jnp.int32, 16)
        attvs = [attb[pl.ds(r * 16, 16)] for r in range(nsl)]

        pltpu.async_copy(xl_h.at[siv.at[0]], xlr[0], semxl[0])
        pltpu.async_copy(xr_h.at[div.at[0]], xrr[0], semxr[0])

        @pl.loop(0, chunks, step=DEPTH)
        def _chunks(t):
            pend = []
            for b in range(DEPTH):
                bn = (b + 1) % DEPTH
                cur = t + b
                pltpu.make_async_copy(
                    xl_h.at[siv.at[0]], xlr[b], semxl[b]).wait()
                pltpu.make_async_copy(
                    xr_h.at[div.at[0]], xrr[b], semxr[b]).wait()

                def _fire():
                    pltpu.async_copy(
                        xl_h.at[siv.at[cur + 1]], xlr[bn], semxl[bn])
                    pltpu.async_copy(
                        xr_h.at[div.at[cur + 1]], xrr[bn], semxr[bn])

                if b == DEPTH - 1:
                    pl.when(t + DEPTH < chunks)(_fire)
                else:
                    _fire()

                base = w * epw + cur * ck

                def group(g, c2):
                    exvec = jnp.zeros((16,), jnp.float32)
                    for j in range(16):
                        ej = g * 16 + j
                        acc = jnp.zeros((16,), jnp.float32)
                        for r in range(nsl):
                            z = (xlr[b][ej, pl.ds(r * 16, 16)]
                                 + xrr[b][ej, pl.ds(r * 16, 16)])
                            acc = acc + attvs[r] * jnp.maximum(z, 0.2 * z)
                        sacc = jnp.sum(acc)
                        valid = (base + ej) < EDGES
                        exj = jnp.where(valid, jnp.exp(sacc + zero16), zero16)
                        exvec = jnp.where(iota16 == j, exj, exvec)
                    exv[b][pl.ds(g * 16, 16)] = exvec
                    return c2

                lax.fori_loop(0, ngr, group, 0)

                if pend:
                    for d in pend.pop(0):
                        d.wait()
                pend.append((
                    pltpu.async_copy(exv[b], ex_h.at[pl.ds(base, ck)],
                                     semex[b]),
                    pltpu.async_copy(exv[b], s_sh.at[div.at[cur]], semss[b],
                                     add=True)))
            for d in pend.pop(0):
                d.wait()

        plsc.subcore_barrier()
        row0 = ss * ROWS_W
        pltpu.sync_copy(s_sh.at[pl.ds(row0, ROWS_W)],
                        s_h.at[cc, pl.ds(row0, ROWS_W)])

    return _a_body


def _sc_a(xl, xr, srcw, dstw, att, feat, ck, chunks):
    epw = ck * chunks
    mesh = plsc.VectorSubcoreMesh(core_axis_name="c", subcore_axis_name="s")
    kfn = pl.kernel(
        _make_a_body(feat, ck, chunks, epw),
        out_type=[jax.ShapeDtypeStruct((EP,), jnp.float32),
                  jax.ShapeDtypeStruct((NCORE, NP), jnp.float32)],
        mesh=mesh,
        scratch_types=[
            pltpu.VMEM((chunks, ck), jnp.int32),
            pltpu.VMEM((chunks, ck), jnp.int32),
        ] + [pltpu.VMEM((ck, feat), jnp.float32)] * 6
          + [pltpu.VMEM((ck,), jnp.float32)] * 3
          + [
            pltpu.VMEM((feat,), jnp.float32),
            pltpu.VMEM_SHARED((NP,), jnp.float32),
        ] + [pltpu.SemaphoreType.DMA] * 12,
        compiler_params=_SC_PARAMS,
    )
    return kfn(xl, xr, srcw, dstw, att)


def _make_b64_body(base_off):
    def _b64_body(xq_h, srcw_h, dstw_h, ex_h, out_h,
                  siv, div,
                  idx0, idx1, idx2, xlr0, xlr1, xlr2, exv0, exv1, exv2,
                  out_sh, *sems):
        idxb = [idx0, idx1, idx2]
        xlr = [xlr0, xlr1, xlr2]
        exv = [exv0, exv1, exv2]
        semg = sems[0:3]
        seme = sems[3:6]
        semsc = sems[6:9]
        cc = lax.axis_index("c")
        ss = lax.axis_index("s")
        hibase = base_off + cc * NP

        pltpu.sync_copy(srcw_h.at[ss], siv)
        pltpu.sync_copy(dstw_h.at[ss], div)

        _zero_vmem_rows(xlr0, C, 64)
        for k in range(ROWS_W // C):
            pltpu.sync_copy(xlr0, out_sh.at[pl.ds(ss * ROWS_W + k * C, C)])
        plsc.subcore_barrier()

        iota16 = lax.iota(jnp.int32, 16)

        def _mkidx(t, b):
            for k in range(C // 16):
                idxb[b][pl.ds(k * 16, 16)] = (
                    siv[t, pl.ds(k * 16, 16)] + hibase)

        def _fire(t, b):
            pltpu.async_copy(xq_h.at[idxb[b]], xlr[b], semg[b])
            pltpu.async_copy(ex_h.at[pl.ds(ss * EPB + t * C, C)],
                             exv[b], seme[b])

        _mkidx(0, 0)
        _fire(0, 0)

        @pl.loop(0, CHUNKS_B, step=DEPTH)
        def _chunks(t):
            pend = []
            for b in range(DEPTH):
                bn = (b + 1) % DEPTH
                cur = t + b
                pltpu.make_async_copy(
                    xq_h.at[idxb[b]], xlr[b], semg[b]).wait()
                pltpu.make_async_copy(
                    ex_h.at[pl.ds(0, C)], exv[b], seme[b]).wait()

                def _fire_next():
                    _mkidx(cur + 1, bn)
                    _fire(cur + 1, bn)

                if b == DEPTH - 1:
                    pl.when(t + DEPTH < CHUNKS_B)(_fire_next)
                else:
                    _fire_next()

                def group(g, c2):
                    ex16 = exv[b][pl.ds(g * 16, 16)]
                    for j in range(16):
                        ej = g * 16 + j
                        aj = jnp.sum(jnp.where(iota16 == j, ex16, 0.0))
                        for r in range(4):
                            xlr[b][ej, pl.ds(r * 16, 16)] = (
                                xlr[b][ej, pl.ds(r * 16, 16)] * aj)
                    return c2

                lax.fori_loop(0, C // 16, group, 0)

                if pend:
                    pend.pop(0).wait()
                pend.append(
                    pltpu.async_copy(xlr[b], out_sh.at[div.at[cur]],
                                     semsc[b], add=True))
            pend.pop(0).wait()

        plsc.subcore_barrier()
        row0 = ss * ROWS_W
        pltpu.sync_copy(out_sh.at[pl.ds(row0, ROWS_W)],
                        out_h.at[cc, pl.ds(row0, ROWS_W)])

    return _b64_body


def _sc_b64(xq, srcw, dstw, ex, base_off):
    mesh = plsc.VectorSubcoreMesh(core_axis_name="c", subcore_axis_name="s")
    kfn = pl.kernel(
        _make_b64_body(base_off),
        out_type=[jax.ShapeDtypeStruct((NCORE, NP, 64), jnp.float32)],
        mesh=mesh,
        scratch_types=[
            pltpu.VMEM((CHUNKS_B, C), jnp.int32),
            pltpu.VMEM((CHUNKS_B, C), jnp.int32),
        ] + [pltpu.VMEM((C,), jnp.int32)] * 3
          + [pltpu.VMEM((C, 64), jnp.float32)] * 3
          + [pltpu.VMEM((C,), jnp.float32)] * 3
          + [pltpu.VMEM_SHARED((NP, 64), jnp.float32)]
          + [pltpu.SemaphoreType.DMA] * 9,
        compiler_params=_SC_PARAMS,
    )
    return kfn(xq, srcw, dstw, ex)[0]


def _t0_body(x_ref, wl_ref, wr_ref, xl_ref, xlq_ref, xr_ref):
    xv = x_ref[...]
    hl = jnp.dot(xv, wl_ref[...], preferred_element_type=jnp.float32)
    xl_ref[...] = hl
    xlq_ref[0] = hl[:, :64]
    xlq_ref[1] = hl[:, 64:]
    xr_ref[...] = jnp.dot(xv, wr_ref[...], preferred_element_type=jnp.float32)


def _t0(xp, wl, wr):
    return pl.pallas_call(
        _t0_body,
        grid=(NP // BM,),
        in_specs=[
            pl.BlockSpec((BM, 128), lambda i: (i, 0)),
            pl.BlockSpec((128, 128), lambda i: (0, 0)),
            pl.BlockSpec((128, 128), lambda i: (0, 0)),
        ],
        out_specs=[
            pl.BlockSpec((BM, 128), lambda i: (i, 0)),
            pl.BlockSpec((2, BM, 64), lambda i: (0, i, 0)),
            pl.BlockSpec((BM, 128), lambda i: (i, 0)),
        ],
        out_shape=[jax.ShapeDtypeStruct((NP, 128), jnp.float32),
                   jax.ShapeDtypeStruct((2, NP, 64), jnp.float32),
                   jax.ShapeDtypeStruct((NP, 128), jnp.float32)],
    )(xp, wl, wr)


def _tmid_body(p_ref, s_ref, b_ref, wl_ref, wr_ref, xl_ref, xlq_ref, xr_ref):
    p = p_ref[...]
    sv = s_ref[...]
    inv = 1.0 / (sv[0] + sv[1] + EPSILON)
    h = jnp.maximum(
        jnp.concatenate([p[0], p[1]], axis=1) * inv[:, None] + b_ref[...],
        0.0)
    hl = jnp.dot(h, wl_ref[...], preferred_element_type=jnp.float32)
    xl_ref[...] = hl
    xlq_ref[0] = hl[:, :64]
    xlq_ref[1] = hl[:, 64:]
    xr_ref[...] = jnp.dot(h, wr_ref[...], preferred_element_type=jnp.float32)


def _tmid(pacc, sacc, b, wl, wr):
    return pl.pallas_call(
        _tmid_body,
        grid=(NP // BM,),
        in_specs=[
            pl.BlockSpec((2, BM, 64), lambda i: (0, i, 0)),
            pl.BlockSpec((2, BM), lambda i: (0, i)),
            pl.BlockSpec((1, 128), lambda i: (0, 0)),
            pl.BlockSpec((128, 128), lambda i: (0, 0)),
            pl.BlockSpec((128, 128), lambda i: (0, 0)),
        ],
        out_specs=[
            pl.BlockSpec((BM, 128), lambda i: (i, 0)),
            pl.BlockSpec((2, BM, 64), lambda i: (0, i, 0)),
            pl.BlockSpec((BM, 128), lambda i: (i, 0)),
        ],
        out_shape=[jax.ShapeDtypeStruct((NP, 128), jnp.float32),
                   jax.ShapeDtypeStruct((2, NP, 64), jnp.float32),
                   jax.ShapeDtypeStruct((NP, 128), jnp.float32)],
    )(pacc, sacc, b.reshape(1, 128), wl, wr)


def _t2_body(p_ref, s_ref, b_ref, wl_ref, wr_ref,
             xlf_ref, xlq_ref, xrf_ref):
    p = p_ref[...]
    sv = s_ref[...]
    inv = 1.0 / (sv[0] + sv[1] + EPSILON)
    h = jnp.maximum(
        jnp.concatenate([p[0], p[1]], axis=1) * inv[:, None] + b_ref[...],
        0.0)
    hl = jnp.dot(h, wl_ref[...], preferred_element_type=jnp.float32)
    xlf_ref[...] = hl
    for q in range(4):
        xlq_ref[q] = hl[:, q * 64:(q + 1) * 64]
    xrf_ref[...] = jnp.dot(h, wr_ref[...], preferred_element_type=jnp.float32)


def _t2(pacc, sacc, b, wl, wr):
    return pl.pallas_call(
        _t2_body,
        grid=(NP // BM,),
        in_specs=[
            pl.BlockSpec((2, BM, 64), lambda i: (0, i, 0)),
            pl.BlockSpec((2, BM), lambda i: (0, i)),
            pl.BlockSpec((1, 128), lambda i: (0, 0)),
            pl.BlockSpec((128, 256), lambda i: (0, 0)),
            pl.BlockSpec((128, 256), lambda i: (0, 0)),
        ],
        out_specs=[
            pl.BlockSpec((BM, 256), lambda i: (i, 0)),
            pl.BlockSpec((4, BM, 64), lambda i: (0, i, 0)),
            pl.BlockSpec((BM, 256), lambda i: (i, 0)),
        ],
        out_shape=[jax.ShapeDtypeStruct((NP, 256), jnp.float32),
                   jax.ShapeDtypeStruct((4, NP, 64), jnp.float32),
                   jax.ShapeDtypeStruct((NP, 256), jnp.float32)],
    )(pacc, sacc, b.reshape(1, 128), wl, wr)


def _t3_body(pa_ref, pb_ref, s_ref, b_ref, wc_ref, bc_ref, o_ref):
    pa = pa_ref[...]
    pb = pb_ref[...]
    sv = s_ref[...]
    inv = 1.0 / (sv[0] + sv[1] + EPSILON)
    h = (jnp.concatenate([pa[0], pa[1], pb[0], pb[1]], axis=1)
         * inv[:, None] + b_ref[...])
    o_ref[...] = jnp.dot(h, wc_ref[...],
                         preferred_element_type=jnp.float32) + bc_ref[...]


def _t3(pa, pb, sacc, b, wc, bc):
    return pl.pallas_call(
        _t3_body,
        grid=(NP // BM,),
        in_specs=[
            pl.BlockSpec((2, BM, 64), lambda i: (0, i, 0)),
            pl.BlockSpec((2, BM, 64), lambda i: (0, i, 0)),
            pl.BlockSpec((2, BM), lambda i: (0, i)),
            pl.BlockSpec((1, 256), lambda i: (0, 0)),
            pl.BlockSpec((256, 40), lambda i: (0, 0)),
            pl.BlockSpec((1, 40), lambda i: (0, 0)),
        ],
        out_specs=pl.BlockSpec((BM, 40), lambda i: (i, 0)),
        out_shape=jax.ShapeDtypeStruct((NP, 40), jnp.float32),
    )(pa, pb, sacc, b.reshape(1, 256), wc, bc.reshape(1, 40))


def kernel(x, edge_index, Wl1, Wr1, att1, b1, Wl2, Wr2, att2, b2,
           Wl3, Wr3, att3, b3, Wc, bc):
    xp = jnp.pad(x, ((0, NP - N), (0, 0)))
    loops = jnp.arange(N, dtype=jnp.int32)
    padi = jnp.arange(EP - EDGES, dtype=jnp.int32) % N
    src = jnp.concatenate([edge_index[0], loops, padi])
    dst = jnp.concatenate([edge_index[1], loops, padi])
    src_a = src.reshape(NW, CHUNKS_A, C)
    dst_a = dst.reshape(NW, CHUNKS_A, C)
    src_a3 = src.reshape(NW, CHUNKS_A3, C3)
    dst_a3 = dst.reshape(NW, CHUNKS_A3, C3)
    src_b = src.reshape(NSUB, CHUNKS_B, C)
    dst_b = dst.reshape(NSUB, CHUNKS_B, C)

    xl1, xl1q, xr1 = _t0(xp, Wl1, Wr1)
    ex1, s1 = _sc_a(xl1, xr1, src_a, dst_a, att1, 128, C, CHUNKS_A)
    p1 = _sc_b64(xl1q.reshape(2 * NP, 64), src_b, dst_b, ex1, 0)
    xl2, xl2q, xr2 = _tmid(p1, s1, b1, Wl2, Wr2)
    ex2, s2 = _sc_a(xl2, xr2, src_a, dst_a, att2, 128, C, CHUNKS_A)
    p2 = _sc_b64(xl2q.reshape(2 * NP, 64), src_b, dst_b, ex2, 0)
    xl3f, xl3q, xr3f = _t2(p2, s2, b2, Wl3, Wr3)
    ex3, s3 = _sc_a(xl3f, xr3f, src_a3, dst_a3, att3, 256, C3, CHUNKS_A3)
    xl3qf = xl3q.reshape(4 * NP, 64)
    p3a = _sc_b64(xl3qf, src_b, dst_b, ex3, 0)
    p3b = _sc_b64(xl3qf, src_b, dst_b, ex3, 2 * NP)
    out = _t3(p3a, p3b, s3, b3, Wc, bc)
    return out[:N]


# speedup vs baseline: 1.0639x; 1.0639x over previous
import functools

import jax
import jax.numpy as jnp
from jax import lax
from jax.experimental import pallas as pl
from jax.experimental.pallas import tpu as pltpu
import jax.experimental.pallas.tpu_sc as plsc

N = 10000
NP = 10240
E = 320000
EDGES = E + N
C = 128
C3 = 64
NSUB = 16
NCORE = 2
NW = NCORE * NSUB
DEPTH = 3
CHUNKS_A = 84
EP = NW * C * CHUNKS_A
EPW = EP // NW
CHUNKS_A3 = EPW // C3
CHUNKS_B = EP // (NSUB * C)
EPB = EP // NSUB
ROWS_W = NP // NSUB
EPSILON = 1e-16
BM = 1280

_SC_PARAMS = pltpu.CompilerParams(needs_layout_passes=False,
                                  use_tc_tiling_on_sc=False)


def _zero_vmem_rows(buf, rows, width):
    zero16 = jnp.zeros((16,), jnp.float32)

    def _z(i, carry):
        for k in range(width // 16):
            buf[i, pl.ds(k * 16, 16)] = zero16
        return carry

    lax.fori_loop(0, rows, _z, 0)


def _make_a_body(feat, ck, chunks, epw):
    nsl = feat // 16
    ngr = ck // 16

    def _a_body(xl_h, xr_h, srcw_h, dstw_h, att_h, ex_h, s_h,
                siv, div,
                xlr0, xlr1, xlr2, exv0, exv1, exv2,
                attb, s_sh, *sems):
        xlr = [xlr0, xlr1, xlr2]
        exv = [exv0, exv1, exv2]
        semxl = sems[0:3]
        semxr = sems[3:6]
        semex = sems[6:9]
        semss = sems[9:12]
        cc = lax.axis_index("c")
        ss = lax.axis_index("s")
        w = cc * NSUB + ss
        zero16 = jnp.zeros((16,), jnp.float32)

        pltpu.sync_copy(srcw_h.at[w], siv)
        pltpu.sync_copy(dstw_h.at[w], div)

        for k in range(ngr):
            exv0[pl.ds(k * 16, 16)] = zero16
        for k in range(ROWS_W // ck):
            pltpu.sync_copy(exv0, s_sh.at[pl.ds(ss * ROWS_W + k * ck, ck)])
        pltpu.sync_copy(att_h, attb)
        plsc.subcore_barrier()

        iota16 = lax.iota(jnp.int32, 16)
        attvs = [attb[pl.ds(r * 16, 16)] for r in range(nsl)]

        pltpu.async_copy(xl_h.at[siv.at[0]], xlr[0], semxl[0])
        pltpu.make_async_copy(xl_h.at[siv.at[0]], xlr[0], semxl[0]).wait()
        pltpu.async_copy(xr_h.at[div.at[0]], xlr[0], semxr[0], add=True)
        pltpu.async_copy(xl_h.at[siv.at[1]], xlr[1], semxl[1])

        @pl.loop(0, chunks, step=DEPTH)
        def _chunks(t):
            pend = []
            for b in range(DEPTH):
                bn = (b + 1) % DEPTH
                bnn = (b + 2) % DEPTH
                cur = t + b

                def _fire_add():
                    pltpu.make_async_copy(
                        xl_h.at[siv.at[0]], xlr[bn], semxl[bn]).wait()
                    pltpu.async_copy(
                        xr_h.at[div.at[cur + 1]], xlr[bn], semxr[bn],
                        add=True)

                if b == DEPTH - 1:
                    pl.when(t + DEPTH < chunks)(_fire_add)
                else:
                    _fire_add()

                def _fire_xl():
                    pltpu.async_copy(
                        xl_h.at[siv.at[cur + 2]], xlr[bnn], semxl[bnn])

                if b >= DEPTH - 2:
                    pl.when(t + b + 2 < chunks)(_fire_xl)
                else:
                    _fire_xl()

                pltpu.make_async_copy(
                    xr_h.at[div.at[0]], xlr[b], semxr[b]).wait()

                base = w * epw + cur * ck

                def group(g, c2):
                    exvec = jnp.zeros((16,), jnp.float32)
                    for j in range(16):
                        ej = g * 16 + j
                        acc = jnp.zeros((16,), jnp.float32)
                        for r in range(nsl):
                            z = xlr[b][ej, pl.ds(r * 16, 16)]
                            acc = acc + attvs[r] * jnp.maximum(z, 0.2 * z)
                        sacc = jnp.sum(acc)
                        valid = (base + ej) < EDGES
                        exj = jnp.where(valid, jnp.exp(sacc + zero16), zero16)
                        exvec = jnp.where(iota16 == j, exj, exvec)
                    exv[b][pl.ds(g * 16, 16)] = exvec
                    return c2

                lax.fori_loop(0, ngr, group, 0)

                if pend:
                    for d in pend.pop(0):
                        d.wait()
                pend.append((
                    pltpu.async_copy(exv[b], ex_h.at[pl.ds(base, ck)],
                                     semex[b]),
                    pltpu.async_copy(exv[b], s_sh.at[div.at[cur]], semss[b],
                                     add=True)))
            for d in pend.pop(0):
                d.wait()

        plsc.subcore_barrier()
        row0 = ss * ROWS_W
        pltpu.sync_copy(s_sh.at[pl.ds(row0, ROWS_W)],
                        s_h.at[cc, pl.ds(row0, ROWS_W)])

    return _a_body


def _sc_a(xl, xr, srcw, dstw, att, feat, ck, chunks):
    epw = ck * chunks
    mesh = plsc.VectorSubcoreMesh(core_axis_name="c", subcore_axis_name="s")
    kfn = pl.kernel(
        _make_a_body(feat, ck, chunks, epw),
        out_type=[jax.ShapeDtypeStruct((EP,), jnp.float32),
                  jax.ShapeDtypeStruct((NCORE, NP), jnp.float32)],
        mesh=mesh,
        scratch_types=[
            pltpu.VMEM((chunks, ck), jnp.int32),
            pltpu.VMEM((chunks, ck), jnp.int32),
        ] + [pltpu.VMEM((ck, feat), jnp.float32)] * 3
          + [pltpu.VMEM((ck,), jnp.float32)] * 3
          + [
            pltpu.VMEM((feat,), jnp.float32),
            pltpu.VMEM_SHARED((NP,), jnp.float32),
        ] + [pltpu.SemaphoreType.DMA] * 12,
        compiler_params=_SC_PARAMS,
    )
    return kfn(xl, xr, srcw, dstw, att)


def _make_b64_body(base_off):
    def _b64_body(xq_h, srcw_h, dstw_h, ex_h, out_h,
                  siv, div,
                  idx0, idx1, idx2, xlr0, xlr1, xlr2, exv0, exv1, exv2,
                  out_sh, *sems):
        idxb = [idx0, idx1, idx2]
        xlr = [xlr0, xlr1, xlr2]
        exv = [exv0, exv1, exv2]
        semg = sems[0:3]
        seme = sems[3:6]
        semsc = sems[6:9]
        cc = lax.axis_index("c")
        ss = lax.axis_index("s")
        hibase = base_off + cc * NP

        pltpu.sync_copy(srcw_h.at[ss], siv)
        pltpu.sync_copy(dstw_h.at[ss], div)

        _zero_vmem_rows(xlr0, C, 64)
        for k in range(ROWS_W // C):
            pltpu.sync_copy(xlr0, out_sh.at[pl.ds(ss * ROWS_W + k * C, C)])
        plsc.subcore_barrier()

        iota16 = lax.iota(jnp.int32, 16)

        def _mkidx(t, b):
            for k in range(C // 16):
                idxb[b][pl.ds(k * 16, 16)] = (
                    siv[t, pl.ds(k * 16, 16)] + hibase)

        def _fire(t, b):
            pltpu.async_copy(xq_h.at[idxb[b]], xlr[b], semg[b])
            pltpu.async_copy(ex_h.at[pl.ds(ss * EPB + t * C, C)],
                             exv[b], seme[b])

        _mkidx(0, 0)
        _fire(0, 0)

        @pl.loop(0, CHUNKS_B, step=DEPTH)
        def _chunks(t):
            pend = []
            for b in range(DEPTH):
                bn = (b + 1) % DEPTH
                cur = t + b
                pltpu.make_async_copy(
                    xq_h.at[idxb[b]], xlr[b], semg[b]).wait()
                pltpu.make_async_copy(
                    ex_h.at[pl.ds(0, C)], exv[b], seme[b]).wait()

                def _fire_next():
                    _mkidx(cur + 1, bn)
                    _fire(cur + 1, bn)

                if b == DEPTH - 1:
                    pl.when(t + DEPTH < CHUNKS_B)(_fire_next)
                else:
                    _fire_next()

                def group(g, c2):
                    ex16 = exv[b][pl.ds(g * 16, 16)]
                    for j in range(16):
                        ej = g * 16 + j
                        aj = jnp.sum(jnp.where(iota16 == j, ex16, 0.0))
                        for r in range(4):
                            xlr[b][ej, pl.ds(r * 16, 16)] = (
                                xlr[b][ej, pl.ds(r * 16, 16)] * aj)
                    return c2

                lax.fori_loop(0, C // 16, group, 0)

                if pend:
                    pend.pop(0).wait()
                pend.append(
                    pltpu.async_copy(xlr[b], out_sh.at[div.at[cur]],
                                     semsc[b], add=True))
            pend.pop(0).wait()

        plsc.subcore_barrier()
        row0 = ss * ROWS_W
        pltpu.sync_copy(out_sh.at[pl.ds(row0, ROWS_W)],
                        out_h.at[cc, pl.ds(row0, ROWS_W)])

    return _b64_body


def _sc_b64(xq, srcw, dstw, ex, base_off):
    mesh = plsc.VectorSubcoreMesh(core_axis_name="c", subcore_axis_name="s")
    kfn = pl.kernel(
        _make_b64_body(base_off),
        out_type=[jax.ShapeDtypeStruct((NCORE, NP, 64), jnp.float32)],
        mesh=mesh,
        scratch_types=[
            pltpu.VMEM((CHUNKS_B, C), jnp.int32),
            pltpu.VMEM((CHUNKS_B, C), jnp.int32),
        ] + [pltpu.VMEM((C,), jnp.int32)] * 3
          + [pltpu.VMEM((C, 64), jnp.float32)] * 3
          + [pltpu.VMEM((C,), jnp.float32)] * 3
          + [pltpu.VMEM_SHARED((NP, 64), jnp.float32)]
          + [pltpu.SemaphoreType.DMA] * 9,
        compiler_params=_SC_PARAMS,
    )
    return kfn(xq, srcw, dstw, ex)[0]


def _t0_body(x_ref, wl_ref, wr_ref, xl_ref, xlq_ref, xr_ref):
    xv = x_ref[...]
    hl = jnp.dot(xv, wl_ref[...], preferred_element_type=jnp.float32)
    xl_ref[...] = hl
    xlq_ref[0] = hl[:, :64]
    xlq_ref[1] = hl[:, 64:]
    xr_ref[...] = jnp.dot(xv, wr_ref[...], preferred_element_type=jnp.float32)


def _t0(xp, wl, wr):
    return pl.pallas_call(
        _t0_body,
        grid=(NP // BM,),
        in_specs=[
            pl.BlockSpec((BM, 128), lambda i: (i, 0)),
            pl.BlockSpec((128, 128), lambda i: (0, 0)),
            pl.BlockSpec((128, 128), lambda i: (0, 0)),
        ],
        out_specs=[
            pl.BlockSpec((BM, 128), lambda i: (i, 0)),
            pl.BlockSpec((2, BM, 64), lambda i: (0, i, 0)),
            pl.BlockSpec((BM, 128), lambda i: (i, 0)),
        ],
        out_shape=[jax.ShapeDtypeStruct((NP, 128), jnp.float32),
                   jax.ShapeDtypeStruct((2, NP, 64), jnp.float32),
                   jax.ShapeDtypeStruct((NP, 128), jnp.float32)],
    )(xp, wl, wr)


def _tmid_body(p_ref, s_ref, b_ref, wl_ref, wr_ref, xl_ref, xlq_ref, xr_ref):
    p = p_ref[...]
    sv = s_ref[...]
    inv = 1.0 / (sv[0] + sv[1] + EPSILON)
    h = jnp.maximum(
        jnp.concatenate([p[0], p[1]], axis=1) * inv[:, None] + b_ref[...],
        0.0)
    hl = jnp.dot(h, wl_ref[...], preferred_element_type=jnp.float32)
    xl_ref[...] = hl
    xlq_ref[0] = hl[:, :64]
    xlq_ref[1] = hl[:, 64:]
    xr_ref[...] = jnp.dot(h, wr_ref[...], preferred_element_type=jnp.float32)


def _tmid(pacc, sacc, b, wl, wr):
    return pl.pallas_call(
        _tmid_body,
        grid=(NP // BM,),
        in_specs=[
            pl.BlockSpec((2, BM, 64), lambda i: (0, i, 0)),
            pl.BlockSpec((2, BM), lambda i: (0, i)),
            pl.BlockSpec((1, 128), lambda i: (0, 0)),
            pl.BlockSpec((128, 128), lambda i: (0, 0)),
            pl.BlockSpec((128, 128), lambda i: (0, 0)),
        ],
        out_specs=[
            pl.BlockSpec((BM, 128), lambda i: (i, 0)),
            pl.BlockSpec((2, BM, 64), lambda i: (0, i, 0)),
            pl.BlockSpec((BM, 128), lambda i: (i, 0)),
        ],
        out_shape=[jax.ShapeDtypeStruct((NP, 128), jnp.float32),
                   jax.ShapeDtypeStruct((2, NP, 64), jnp.float32),
                   jax.ShapeDtypeStruct((NP, 128), jnp.float32)],
    )(pacc, sacc, b.reshape(1, 128), wl, wr)


def _t2_body(p_ref, s_ref, b_ref, wl_ref, wr_ref,
             xlf_ref, xlq_ref, xrf_ref):
    p = p_ref[...]
    sv = s_ref[...]
    inv = 1.0 / (sv[0] + sv[1] + EPSILON)
    h = jnp.maximum(
        jnp.concatenate([p[0], p[1]], axis=1) * inv[:, None] + b_ref[...],
        0.0)
    hl = jnp.dot(h, wl_ref[...], preferred_element_type=jnp.float32)
    xlf_ref[...] = hl
    for q in range(4):
        xlq_ref[q] = hl[:, q * 64:(q + 1) * 64]
    xrf_ref[...] = jnp.dot(h, wr_ref[...], preferred_element_type=jnp.float32)


def _t2(pacc, sacc, b, wl, wr):
    return pl.pallas_call(
        _t2_body,
        grid=(NP // BM,),
        in_specs=[
            pl.BlockSpec((2, BM, 64), lambda i: (0, i, 0)),
            pl.BlockSpec((2, BM), lambda i: (0, i)),
            pl.BlockSpec((1, 128), lambda i: (0, 0)),
            pl.BlockSpec((128, 256), lambda i: (0, 0)),
            pl.BlockSpec((128, 256), lambda i: (0, 0)),
        ],
        out_specs=[
            pl.BlockSpec((BM, 256), lambda i: (i, 0)),
            pl.BlockSpec((4, BM, 64), lambda i: (0, i, 0)),
            pl.BlockSpec((BM, 256), lambda i: (i, 0)),
        ],
        out_shape=[jax.ShapeDtypeStruct((NP, 256), jnp.float32),
                   jax.ShapeDtypeStruct((4, NP, 64), jnp.float32),
                   jax.ShapeDtypeStruct((NP, 256), jnp.float32)],
    )(pacc, sacc, b.reshape(1, 128), wl, wr)


def _t3_body(pa_ref, pb_ref, s_ref, b_ref, wc_ref, bc_ref, o_ref):
    pa = pa_ref[...]
    pb = pb_ref[...]
    sv = s_ref[...]
    inv = 1.0 / (sv[0] + sv[1] + EPSILON)
    h = (jnp.concatenate([pa[0], pa[1], pb[0], pb[1]], axis=1)
         * inv[:, None] + b_ref[...])
    o_ref[...] = jnp.dot(h, wc_ref[...],
                         preferred_element_type=jnp.float32) + bc_ref[...]


def _t3(pa, pb, sacc, b, wc, bc):
    return pl.pallas_call(
        _t3_body,
        grid=(NP // BM,),
        in_specs=[
            pl.BlockSpec((2, BM, 64), lambda i: (0, i, 0)),
            pl.BlockSpec((2, BM, 64), lambda i: (0, i, 0)),
            pl.BlockSpec((2, BM), lambda i: (0, i)),
            pl.BlockSpec((1, 256), lambda i: (0, 0)),
            pl.BlockSpec((256, 40), lambda i: (0, 0)),
            pl.BlockSpec((1, 40), lambda i: (0, 0)),
        ],
        out_specs=pl.BlockSpec((BM, 40), lambda i: (i, 0)),
        out_shape=jax.ShapeDtypeStruct((NP, 40), jnp.float32),
    )(pa, pb, sacc, b.reshape(1, 256), wc, bc.reshape(1, 40))


def kernel(x, edge_index, Wl1, Wr1, att1, b1, Wl2, Wr2, att2, b2,
           Wl3, Wr3, att3, b3, Wc, bc):
    xp = jnp.pad(x, ((0, NP - N), (0, 0)))
    loops = jnp.arange(N, dtype=jnp.int32)
    padi = jnp.arange(EP - EDGES, dtype=jnp.int32) % N
    src = jnp.concatenate([edge_index[0], loops, padi])
    dst = jnp.concatenate([edge_index[1], loops, padi])
    src_a = src.reshape(NW, CHUNKS_A, C)
    dst_a = dst.reshape(NW, CHUNKS_A, C)
    src_a3 = src.reshape(NW, CHUNKS_A3, C3)
    dst_a3 = dst.reshape(NW, CHUNKS_A3, C3)
    src_b = src.reshape(NSUB, CHUNKS_B, C)
    dst_b = dst.reshape(NSUB, CHUNKS_B, C)

    xl1, xl1q, xr1 = _t0(xp, Wl1, Wr1)
    ex1, s1 = _sc_a(xl1, xr1, src_a, dst_a, att1, 128, C, CHUNKS_A)
    p1 = _sc_b64(xl1q.reshape(2 * NP, 64), src_b, dst_b, ex1, 0)
    xl2, xl2q, xr2 = _tmid(p1, s1, b1, Wl2, Wr2)
    ex2, s2 = _sc_a(xl2, xr2, src_a, dst_a, att2, 128, C, CHUNKS_A)
    p2 = _sc_b64(xl2q.reshape(2 * NP, 64), src_b, dst_b, ex2, 0)
    xl3f, xl3q, xr3f = _t2(p2, s2, b2, Wl3, Wr3)
    ex3, s3 = _sc_a(xl3f, xr3f, src_a3, dst_a3, att3, 256, C3, CHUNKS_A3)
    xl3qf = xl3q.reshape(4 * NP, 64)
    p3a = _sc_b64(xl3qf, src_b, dst_b, ex3, 0)
    p3b = _sc_b64(xl3qf, src_b, dst_b, ex3, 2 * NP)
    out = _t3(p3a, p3b, s3, b3, Wc, bc)
    return out[:N]
